# Initial kernel scaffold; baseline (speedup 1.0000x reference)
import jax
import jax.numpy as jnp
from jax.experimental import pallas as pl


def kernel(x, edge_index, edge_weight, W1, b1, W2, b2):
    raise NotImplementedError("write your pallas kernel here")


# speedup vs baseline: 10.3372x; 10.3372x over previous
import functools

import jax
import jax.numpy as jnp
from jax import lax
from jax.experimental import pallas as pl
from jax.experimental.pallas import tpu as pltpu
from jax.experimental.pallas import tpu_sc as plsc

N = 10000
E = 320000
NC = 2
NS = 16
NW = NC * NS
EPW = E // NW
KB = 80
NB = EPW // KB
SLAB = 632


def _slab_base(sid):
    return jnp.minimum(sid * SLAB, N - SLAB)

_MESH = plsc.VectorSubcoreMesh(core_axis_name="c", subcore_axis_name="s")
_SC_PARAMS = pltpu.CompilerParams(use_tc_tiling_on_sc=False)


def _deg_body(dst_hbm, w_hbm, z_hbm, deg_hbm, dst_v, w_v, rows_v, degsh, sem):
    cid = lax.axis_index("c")
    sid = lax.axis_index("s")
    wid = cid * NS + sid
    sb = _slab_base(sid)
    pltpu.sync_copy(z_hbm.at[pl.ds(sb, SLAB)], degsh.at[pl.ds(sb, SLAB)])
    plsc.subcore_barrier()

    def batch(g, carry):
        base = wid * EPW + g * KB
        pltpu.sync_copy(dst_hbm.at[pl.ds(base, KB)], dst_v)
        pltpu.sync_copy(w_hbm.at[pl.ds(base, KB)], w_v)

        for g in range(KB // 16):
            wv = w_v[pl.ds(g * 16, 16)]
            for j in range(16):
                rows_v[g * 16 + j, :] = jnp.broadcast_to(wv[j], (16,))
        pltpu.sync_copy(rows_v, degsh.at[dst_v], add=True)
        return carry

    lax.fori_loop(0, NB, batch, 0)
    plsc.subcore_barrier()
    pltpu.sync_copy(degsh.at[pl.ds(sb, SLAB)],
                    deg_hbm.at[pl.ds(cid * N + sb, SLAB)])


_deg_kernel = pl.kernel(
    _deg_body,
    out_type=jax.ShapeDtypeStruct((2 * N, 16), jnp.float32),
    mesh=_MESH,
    compiler_params=_SC_PARAMS,
    scratch_types=[
        pltpu.VMEM((KB,), jnp.int32),
        pltpu.VMEM((KB,), jnp.float32),
        pltpu.VMEM((KB, 16), jnp.float32),
        pltpu.VMEM_SHARED((N, 16), jnp.float32),
        pltpu.SemaphoreType.DMA,
    ],
)


def _msg_body(F, src_hbm, dst_hbm, w_hbm, ht_hbm, z_hbm, out_hbm,
              src_v, dst_v, w_v, rows_v, accsh, sem):
    cid = lax.axis_index("c")
    sid = lax.axis_index("s")
    wid = cid * NS + sid
    sb = _slab_base(sid)
    pltpu.sync_copy(z_hbm.at[pl.ds(sb, SLAB)], accsh.at[pl.ds(sb, SLAB)])
    plsc.subcore_barrier()

    def batch(g, carry):
        base = wid * EPW + g * KB
        pltpu.sync_copy(src_hbm.at[pl.ds(base, KB)], src_v)
        pltpu.sync_copy(dst_hbm.at[pl.ds(base, KB)], dst_v)
        pltpu.sync_copy(w_hbm.at[pl.ds(base, KB)], w_v)
        pltpu.async_copy(ht_hbm.at[src_v], rows_v, sem).wait()

        for g in range(KB // 16):
            wv = w_v[pl.ds(g * 16, 16)]
            for j in range(16):
                k = g * 16 + j
                s = wv[j]
                for f in range(F // 16):
                    sl = pl.ds(f * 16, 16)
                    rows_v[k, sl] = rows_v[k, sl] * s
        pltpu.sync_copy(rows_v, accsh.at[dst_v], add=True)
        return carry

    lax.fori_loop(0, NB, batch, 0)
    plsc.subcore_barrier()
    pltpu.sync_copy(accsh.at[pl.ds(sb, SLAB)],
                    out_hbm.at[pl.ds(cid * N + sb, SLAB)])


def _make_msg_kernel(F):
    return pl.kernel(
        functools.partial(_msg_body, F),
        out_type=jax.ShapeDtypeStruct((2 * N, F), jnp.float32),
        mesh=_MESH,
        compiler_params=_SC_PARAMS,
        scratch_types=[
            pltpu.VMEM((KB,), jnp.int32),
            pltpu.VMEM((KB,), jnp.int32),
            pltpu.VMEM((KB,), jnp.float32),
            pltpu.VMEM((KB, F), jnp.float32),
            pltpu.VMEM_SHARED((N, F), jnp.float32),
            pltpu.SemaphoreType.DMA,
        ],
    )


_msg_kernel_128 = _make_msg_kernel(128)
_msg_kernel_64 = _make_msg_kernel(64)


_R = 1000
_G = N // _R


def _tc1_body(x_ref, w_ref, dga_ref, dgb_ref, ht_ref, dis_ref):
    dis = lax.rsqrt(dga_ref[:, :1] + dgb_ref[:, :1] + 1.0)
    ht_ref[...] = jnp.dot(x_ref[...], w_ref[...],
                          preferred_element_type=jnp.float32) * dis
    dis_ref[...] = dis


def _tc1(x, W1, deg_parts):
    return pl.pallas_call(
        _tc1_body,
        grid=(_G,),
        in_specs=[
            pl.BlockSpec((_R, 128), lambda i: (i, 0)),
            pl.BlockSpec((128, 128), lambda i: (0, 0)),
            pl.BlockSpec((_R, 16), lambda i: (i, 0)),
            pl.BlockSpec((_R, 16), lambda i: (i + _G, 0)),
        ],
        out_specs=[
            pl.BlockSpec((_R, 128), lambda i: (i, 0)),
            pl.BlockSpec((_R, 1), lambda i: (i, 0)),
        ],
        out_shape=[
            jax.ShapeDtypeStruct((N, 128), jnp.float32),
            jax.ShapeDtypeStruct((N, 1), jnp.float32),
        ],
    )(x, W1, deg_parts, deg_parts)


def _tc2_body(a0_ref, a1_ref, ht_ref, dis_ref, b_ref, w_ref, out_ref):
    dis = dis_ref[...]
    x1 = dis * (a0_ref[...] + a1_ref[...] + ht_ref[...]) + b_ref[...]
    out_ref[...] = jnp.dot(x1, w_ref[...],
                           preferred_element_type=jnp.float32) * dis


def _tc2(acc1, ht1, dis, b1, W2):
    return pl.pallas_call(
        _tc2_body,
        grid=(_G,),
        in_specs=[
            pl.BlockSpec((_R, 128), lambda i: (i, 0)),
            pl.BlockSpec((_R, 128), lambda i: (i + _G, 0)),
            pl.BlockSpec((_R, 128), lambda i: (i, 0)),
            pl.BlockSpec((_R, 1), lambda i: (i, 0)),
            pl.BlockSpec((1, 128), lambda i: (0, 0)),
            pl.BlockSpec((128, 64), lambda i: (0, 0)),
        ],
        out_specs=pl.BlockSpec((_R, 64), lambda i: (i, 0)),
        out_shape=jax.ShapeDtypeStruct((N, 64), jnp.float32),
    )(acc1, acc1, ht1, dis, b1, W2)


def _tc3_body(a0_ref, a1_ref, ht_ref, dis_ref, b_ref, out_ref):
    out_ref[...] = (dis_ref[...] * (a0_ref[...] + a1_ref[...] + ht_ref[...])
                    + b_ref[...])


def _tc3(acc2, ht2, dis, b2):
    return pl.pallas_call(
        _tc3_body,
        grid=(_G,),
        in_specs=[
            pl.BlockSpec((_R, 64), lambda i: (i, 0)),
            pl.BlockSpec((_R, 64), lambda i: (i + _G, 0)),
            pl.BlockSpec((_R, 64), lambda i: (i, 0)),
            pl.BlockSpec((_R, 1), lambda i: (i, 0)),
            pl.BlockSpec((1, 64), lambda i: (0, 0)),
        ],
        out_specs=pl.BlockSpec((_R, 64), lambda i: (i, 0)),
        out_shape=jax.ShapeDtypeStruct((N, 64), jnp.float32),
    )(acc2, acc2, ht2, dis, b2)


def kernel(x, edge_index, edge_weight, W1, b1, W2, b2):
    src = edge_index[0].astype(jnp.int32)
    dst = edge_index[1].astype(jnp.int32)
    w = edge_weight.astype(jnp.float32)
    z16 = jnp.zeros((N, 16), jnp.float32)
    z128 = jnp.zeros((N, 128), jnp.float32)
    z64 = jnp.zeros((N, 64), jnp.float32)

    deg_parts = _deg_kernel(dst, w, z16)
    ht1, dis = _tc1(x, W1, deg_parts)
    acc1 = _msg_kernel_128(src, dst, w, ht1, z128)
    ht2 = _tc2(acc1, ht1, dis, b1.reshape(1, -1), W2)
    acc2 = _msg_kernel_64(src, dst, w, ht2, z64)
    return _tc3(acc2, ht2, dis, b2.reshape(1, -1))
